# baseline (speedup 1.0000x reference)
import functools

import jax
import jax.numpy as jnp
from jax import lax
from jax.experimental import pallas as pl
from jax.experimental.pallas import tpu as pltpu
from jax.experimental.pallas import tpu_sc as plsc

SEQ_LEN = 8192
MODEL_DIM = 1024

_NC = 2
_NS = 16
_NW = _NC * _NS

_CHUNK = 16
_NSTEPS = SEQ_LEN // _CHUNK // _NW
_NBUF = 7
_SLACK = 3

_mesh = plsc.VectorSubcoreMesh(core_axis_name="c", subcore_axis_name="s")


@functools.partial(
    pl.kernel,
    mesh=_mesh,
    out_type=jax.ShapeDtypeStruct((SEQ_LEN, MODEL_DIM), jnp.float32),
    scratch_types=[
        pltpu.VMEM((_NBUF, _CHUNK, MODEL_DIM), jnp.float32),
        pltpu.SemaphoreType.DMA((_NBUF,)),
        pltpu.SemaphoreType.DMA((_NBUF,)),
    ],
)
def _sc_copy(table_hbm, out_hbm, buf, sem_in, sem_out):
    wid = lax.axis_index("s") * _NC + lax.axis_index("c")

    def chunk_row(step):
        return (step * _NW + wid) * _CHUNK

    in_cp = [None] * _NSTEPS
    out_cp = [None] * _NSTEPS

    def start_in(step):
        b = step % _NBUF
        return pltpu.async_copy(
            table_hbm.at[pl.ds(chunk_row(step), _CHUNK)],
            buf.at[b],
            sem_in.at[b],
        )

    for step in range(min(_NBUF, _NSTEPS)):
        in_cp[step] = start_in(step)

    for step in range(_NSTEPS):
        b = step % _NBUF
        in_cp[step].wait()
        out_cp[step] = pltpu.async_copy(
            buf.at[b],
            out_hbm.at[pl.ds(chunk_row(step), _CHUNK)],
            sem_out.at[b],
        )
        j = step - _SLACK
        nxt = j + _NBUF
        if j >= 0 and nxt < _NSTEPS:
            out_cp[j].wait()
            in_cp[nxt] = start_in(nxt)

    for step in range(max(0, _NSTEPS - _NBUF), _NSTEPS):
        out_cp[step].wait()


def kernel(x, emb_weight):
    del x
    return _sc_copy(emb_weight)
